# baseline (speedup 1.0000x reference)
import functools

import jax
import jax.numpy as jnp
from jax import lax
from jax.experimental import pallas as pl
from jax.experimental.pallas import tpu as pltpu
from jax.experimental.pallas import tpu_sc as plsc

NUM_CORES = 2
NUM_SUBCORES = 16
NUM_WORKERS = NUM_CORES * NUM_SUBCORES
LANES = 16

BATCH = 16384
EMBED_DIM = 32
NROWS = 1000000
TROWS = NROWS // 4
BROWS = NROWS // 128
BTAIL = BROWS * 128
B_PER_W = BATCH // NUM_WORKERS
SUB = 128
N_SUB = B_PER_W // SUB
N_BLOCKS = SUB // LANES


def _mf_body(uid_hbm, iid_hbm, utab_hbm, itab_hbm, bias_hbm, btail_hbm,
             out_hbm, uidx, iidx, ugrp, igrp, ibg, ucol, icol, bgr, btl,
             out_v, sem, bsem):
    wid = lax.axis_index("s") * NUM_CORES + lax.axis_index("c")
    base = wid * B_PER_W

    pltpu.sync_copy(uid_hbm.at[pl.ds(base, B_PER_W)], uidx)
    pltpu.sync_copy(iid_hbm.at[pl.ds(base, B_PER_W)], iidx)
    tl = pltpu.async_copy(btail_hbm, btl, bsem)

    @pl.loop(0, B_PER_W // LANES)
    def _grp_block(i):
        off = pl.multiple_of(i * LANES, LANES)
        uv = uidx[pl.ds(off, LANES)]
        iv = iidx[pl.ds(off, LANES)]
        ugrp[pl.ds(off, LANES)] = lax.shift_right_logical(uv, 2)
        igrp[pl.ds(off, LANES)] = lax.shift_right_logical(iv, 2)
        ibg[pl.ds(off, LANES)] = jnp.minimum(
            lax.shift_right_logical(iv, 7), BROWS - 1)

    bias_copies = []
    for j in range(N_SUB):
        cds = pl.ds(j * SUB, SUB)
        bias_copies.append(
            pltpu.async_copy(bias_hbm.at[ibg.at[cds]], bgr.at[cds], bsem))
    for cp in bias_copies:
        cp.wait()
    tl.wait()

    @pl.loop(0, N_SUB, unroll=1)
    def _sub_batch(sb):
        k0 = pl.multiple_of(sb * SUB, SUB)
        cds = pl.ds(k0, SUB)
        cu = pltpu.async_copy(utab_hbm.at[ugrp.at[cds]], ucol, sem)
        ci = pltpu.async_copy(itab_hbm.at[igrp.at[cds]], icol, sem)
        cu.wait()
        ci.wait()

        for bi in range(N_BLOCKS):
            b0 = pl.multiple_of(k0 + bi * LANES, LANES)
            uv = uidx[pl.ds(b0, LANES)]
            iv = iidx[pl.ds(b0, LANES)]
            ucl = (uv & 3) * EMBED_DIM
            icl = (iv & 3) * EMBED_DIM
            row16 = lax.iota(jnp.int32, LANES) + bi * LANES
            bmain = plsc.load_gather(bgr, [k0 + row16, iv & 127])
            btail = plsc.load_gather(btl, [jnp.maximum(iv - BTAIL, 0)])
            acc = jnp.where(iv >= BTAIL, btail, bmain)
            for d in range(EMBED_DIM):
                gu = plsc.load_gather(ucol, [row16, ucl + d])
                gi = plsc.load_gather(icol, [row16, icl + d])
                acc = acc + gu * gi
            out_v[pl.ds(b0, LANES)] = acc

    pltpu.sync_copy(out_v, out_hbm.at[pl.ds(base, B_PER_W)])


_mf_call = functools.partial(
    pl.kernel,
    out_type=jax.ShapeDtypeStruct((BATCH,), jnp.float32),
    mesh=plsc.VectorSubcoreMesh(core_axis_name="c", subcore_axis_name="s",
                                num_cores=NUM_CORES,
                                num_subcores=NUM_SUBCORES),
    scratch_types=[
        pltpu.VMEM((B_PER_W,), jnp.int32),
        pltpu.VMEM((B_PER_W,), jnp.int32),
        pltpu.VMEM((B_PER_W,), jnp.int32),
        pltpu.VMEM((B_PER_W,), jnp.int32),
        pltpu.VMEM((B_PER_W,), jnp.int32),
        pltpu.VMEM((SUB, 128), jnp.float32),
        pltpu.VMEM((SUB, 128), jnp.float32),
        pltpu.VMEM((B_PER_W, 128), jnp.float32),
        pltpu.VMEM((NROWS - BROWS * 128,), jnp.float32),
        pltpu.VMEM((B_PER_W,), jnp.float32),
        pltpu.SemaphoreType.DMA,
        pltpu.SemaphoreType.DMA,
    ],
    compiler_params=pltpu.CompilerParams(needs_layout_passes=False),
)(_mf_body)


@jax.jit
def kernel(user_ids, item_ids, user_table, item_table, item_bias_table):
    ut = user_table.reshape(TROWS, 128)
    it = item_table.reshape(TROWS, 128)
    bflat = item_bias_table.reshape(NROWS)
    bt = bflat[:BTAIL].reshape(BROWS, 128)
    btail = bflat[BTAIL:]
    return _mf_call(user_ids.astype(jnp.int32), item_ids.astype(jnp.int32),
                    ut, it, bt, btail)
